# baseline (speedup 1.0000x reference)
import functools
import jax
import jax.numpy as jnp
from jax import lax
from jax.experimental import pallas as pl
from jax.experimental.pallas import tpu as pltpu

_B, _T, _C = 1, 512, 256
_W = 512
_ER, _ES, _K = 16, 2, 2
_EPS = 1.1920929e-07
_BLK = 128
_EPG = 4
_MAXB = _T // _BLK


def _rms(x, g):
    return x * jax.lax.rsqrt(jnp.mean(x * x, axis=-1, keepdims=True) + _EPS) * g


def _gelu(x):
    return 0.5 * x * (1.0 + jax.lax.erf(x * 0.7071067811865476))


def _moe_body(u_ref, cent_ref, sg_ref, rg_ref,
              sW1_ref, sb1_ref, sW2_ref, sb2_ref,
              rW1_ref, rb1_ref, rW2_ref, rb2_ref,
              out_ref, xn_scr, p_scr, g_scr, meta_scr):
    step = pl.program_id(0)
    u = u_ref[...]
    bf = jnp.bfloat16

    @pl.when(step == 0)
    def _init():
        out_ref[...] = u
        xn_scr[...] = _rms(u, rg_ref[...]).astype(bf)

        s = jax.nn.sigmoid(
            jnp.dot(u, cent_ref[...], preferred_element_type=jnp.float32))
        ids = jax.lax.broadcasted_iota(jnp.int32, (_T, _ER), 1)
        denom = jnp.sum(s, axis=1, keepdims=True)
        m1 = jnp.max(s, axis=1, keepdims=True)
        i1 = jnp.min(jnp.where(s == m1, ids, _ER), axis=1, keepdims=True)
        s2 = jnp.where(ids == i1, -jnp.inf, s)
        m2 = jnp.max(s2, axis=1, keepdims=True)
        i2 = jnp.min(jnp.where(s2 == m2, ids, _ER), axis=1, keepdims=True)
        g_scr[...] = jnp.concatenate([m1 / denom, m2 / denom], axis=1)

        O0 = (ids == i1).astype(jnp.float32)
        O1 = (ids == i2).astype(jnp.float32)
        rT = jax.lax.broadcasted_iota(jnp.int32, (_T, _T), 0)
        cT = jax.lax.broadcasted_iota(jnp.int32, (_T, _T), 1)
        Lst = (cT < rT).astype(jnp.float32)
        cums = (jnp.dot(Lst, O0, preferred_element_type=jnp.float32)
                + jnp.dot(Lst, O1, preferred_element_type=jnp.float32))
        ctot = jnp.sum(O0 + O1, axis=0, keepdims=True)
        npad = jnp.floor((ctot + (_BLK - 1)) * (1.0 / _BLK)) * _BLK
        rE = jax.lax.broadcasted_iota(jnp.int32, (_ER, _ER), 0)
        cE = jax.lax.broadcasted_iota(jnp.int32, (_ER, _ER), 1)
        Mex = (rE < cE).astype(jnp.float32)
        offp = jnp.dot(npad, Mex, preferred_element_type=jnp.float32)
        p0 = jnp.sum(O0 * (offp + cums), axis=1, keepdims=True)
        p1 = jnp.sum(O1 * (offp + cums), axis=1, keepdims=True)
        p_scr[...] = jnp.concatenate([p0, p1], axis=1).astype(jnp.int32)

        for ee in range(_ER):
            meta_scr[0, ee] = jnp.sum(offp[:, ee]).astype(jnp.int32)
            meta_scr[1, ee] = jnp.sum(
                npad[:, ee] * (1.0 / _BLK)).astype(jnp.int32)

    @pl.when(step < _ES)
    def _shared():
        xns = _rms(u, sg_ref[...])
        h = _gelu(jnp.dot(xns.astype(bf), sW1_ref[0].astype(bf),
                          preferred_element_type=jnp.float32) + sb1_ref[0])
        out_ref[...] += (jnp.dot(h.astype(bf), sW2_ref[0].astype(bf),
                                 preferred_element_type=jnp.float32)
                         + sb2_ref[0])

    xn = xn_scr[...]
    p0 = p_scr[:, 0:1]
    p1 = p_scr[:, 1:2]
    g0 = g_scr[:, 0:1]
    g1 = g_scr[:, 1:2]
    lane = jax.lax.broadcasted_iota(jnp.int32, (_T, _BLK), 1)

    for sub in range(_EPG):
        e = step * _EPG + sub
        start = meta_scr[0, e]
        nblk = meta_scr[1, e]
        W1 = rW1_ref[sub].astype(bf)
        b1 = rb1_ref[sub]
        W2 = rW2_ref[sub].astype(bf)
        b2 = rb2_ref[sub]
        for j in range(_MAXB):
            @pl.when(j < nblk)
            def _block(j=j, start=start, W1=W1, b1=b1, W2=W2, b2=b2):
                gr = lane + (start + j * _BLK)
                c0 = p0 == gr
                c1 = p1 == gr
                selT = (jnp.where(c0, 1.0, 0.0)
                        + jnp.where(c1, 1.0, 0.0)).astype(bf)
                selg = (jnp.where(c0, g0, 0.0)
                        + jnp.where(c1, g1, 0.0)).astype(bf)
                x = lax.dot_general(selT, xn, (((0,), (0,)), ((), ())),
                                    preferred_element_type=jnp.float32)
                h = _gelu(jnp.dot(x.astype(bf), W1,
                                  preferred_element_type=jnp.float32) + b1)
                y = jnp.dot(h.astype(bf), W2,
                            preferred_element_type=jnp.float32) + b2
                out_ref[...] += jnp.dot(selg, y.astype(bf),
                                        preferred_element_type=jnp.float32)


def kernel(u, shared_W1, shared_b1, shared_W2, shared_b2, shared_g,
           routed_W1, routed_b1, routed_W2, routed_b2, routed_g, centroids):
    u2 = u.reshape(_T, _C)
    nsteps = _ER // _EPG
    out = pl.pallas_call(
        _moe_body,
        grid=(nsteps,),
        in_specs=[
            pl.BlockSpec((_T, _C), lambda e: (0, 0)),
            pl.BlockSpec((_C, _ER), lambda e: (0, 0)),
            pl.BlockSpec((1, _C), lambda e: (0, 0)),
            pl.BlockSpec((1, _C), lambda e: (0, 0)),
            pl.BlockSpec((1, _C, _W), lambda e: (jnp.minimum(e, _ES - 1), 0, 0)),
            pl.BlockSpec((1, 1, _W), lambda e: (jnp.minimum(e, _ES - 1), 0, 0)),
            pl.BlockSpec((1, _W, _C), lambda e: (jnp.minimum(e, _ES - 1), 0, 0)),
            pl.BlockSpec((1, 1, _C), lambda e: (jnp.minimum(e, _ES - 1), 0, 0)),
            pl.BlockSpec((_EPG, _C, _W), lambda e: (e, 0, 0)),
            pl.BlockSpec((_EPG, 1, _W), lambda e: (e, 0, 0)),
            pl.BlockSpec((_EPG, _W, _C), lambda e: (e, 0, 0)),
            pl.BlockSpec((_EPG, 1, _C), lambda e: (e, 0, 0)),
        ],
        out_specs=pl.BlockSpec((_T, _C), lambda e: (0, 0)),
        out_shape=jax.ShapeDtypeStruct((_T, _C), jnp.float32),
        scratch_shapes=[
            pltpu.VMEM((_T, _C), jnp.bfloat16),
            pltpu.VMEM((_T, _K), jnp.int32),
            pltpu.VMEM((_T, _K), jnp.float32),
            pltpu.SMEM((2, _ER), jnp.int32),
        ],
        compiler_params=pltpu.CompilerParams(
            dimension_semantics=("arbitrary",),
        ),
    )(
        u2, centroids,
        shared_g.reshape(1, _C), routed_g.reshape(1, _C),
        shared_W1, shared_b1.reshape(_ES, 1, _W),
        shared_W2, shared_b2.reshape(_ES, 1, _C),
        routed_W1, routed_b1.reshape(_ER, 1, _W),
        routed_W2, routed_b2.reshape(_ER, 1, _C),
    )
    return out.reshape(_B, _T, _C)


# speedup vs baseline: 1.5040x; 1.5040x over previous
import functools
import jax
import jax.numpy as jnp
from jax.experimental import pallas as pl
from jax.experimental.pallas import tpu as pltpu

_B, _T, _C = 1, 512, 256
_W = 512
_ER, _ES, _K = 16, 2, 2
_EPS = 1.1920929e-07


def _rms(x, g):
    return x * jax.lax.rsqrt(jnp.mean(x * x, axis=-1, keepdims=True) + _EPS) * g


def _gelu(x):
    return 0.5 * x * (1.0 + jax.lax.erf(x * 0.7071067811865476))


def _dense_body(u_ref, cent_ref, sg_ref, rg_ref,
                sW1_ref, sb1_ref, sW2_ref, sb2_ref,
                rW1_ref, rb1_ref, rW2_ref, rb2_ref,
                out_ref, g_scr):
    e = pl.program_id(0)
    u = u_ref[...]
    ids = jax.lax.broadcasted_iota(jnp.int32, (_T, _ER), 1)

    @pl.when(e == 0)
    def _init():
        s = jax.nn.sigmoid(
            jnp.dot(u, cent_ref[...], preferred_element_type=jnp.float32))
        denom = jnp.sum(s, axis=1, keepdims=True)
        m1 = jnp.max(s, axis=1, keepdims=True)
        i1 = jnp.min(jnp.where(s == m1, ids, _ER), axis=1, keepdims=True)
        s2 = jnp.where(ids == i1, -jnp.inf, s)
        m2 = jnp.max(s2, axis=1, keepdims=True)
        i2 = jnp.min(jnp.where(s2 == m2, ids, _ER), axis=1, keepdims=True)
        gfull = (jnp.where(ids == i1, m1 / denom, 0.0)
                 + jnp.where(ids == i2, m2 / denom, 0.0))
        g_scr[...] = gfull
        out_ref[...] = u

    bf = jnp.bfloat16

    @pl.when(e < _ES)
    def _shared():
        xn = _rms(u, sg_ref[0, :])
        h = _gelu(jnp.dot(xn.astype(bf), sW1_ref[0].astype(bf),
                          preferred_element_type=jnp.float32)
                  + sb1_ref[0])
        out_ref[...] += (jnp.dot(h.astype(bf), sW2_ref[0].astype(bf),
                                 preferred_element_type=jnp.float32)
                         + sb2_ref[0])

    xn = _rms(u, rg_ref[0, :])
    xnb = xn.astype(bf)
    acc = jnp.zeros((_T, _C), jnp.float32)
    for sub in range(4):
        ee = e * 4 + sub
        h = _gelu(jnp.dot(xnb, rW1_ref[sub].astype(bf),
                          preferred_element_type=jnp.float32)
                  + rb1_ref[sub])
        y = jnp.dot(h.astype(bf), rW2_ref[sub].astype(bf),
                    preferred_element_type=jnp.float32) + rb2_ref[sub]
        gcol = jnp.sum(jnp.where(ids == ee, g_scr[...], 0.0), axis=1,
                       keepdims=True)
        acc = acc + gcol * y
    out_ref[...] += acc


def kernel(u, shared_W1, shared_b1, shared_W2, shared_b2, shared_g,
           routed_W1, routed_b1, routed_W2, routed_b2, routed_g, centroids):
    u2 = u.reshape(_T, _C)
    out = pl.pallas_call(
        _dense_body,
        grid=(_ER // 4,),
        in_specs=[
            pl.BlockSpec((_T, _C), lambda e: (0, 0)),
            pl.BlockSpec((_C, _ER), lambda e: (0, 0)),
            pl.BlockSpec((1, _C), lambda e: (0, 0)),
            pl.BlockSpec((1, _C), lambda e: (0, 0)),
            pl.BlockSpec((1, _C, _W), lambda e: (jnp.minimum(e, _ES - 1), 0, 0)),
            pl.BlockSpec((1, 1, _W), lambda e: (jnp.minimum(e, _ES - 1), 0, 0)),
            pl.BlockSpec((1, _W, _C), lambda e: (jnp.minimum(e, _ES - 1), 0, 0)),
            pl.BlockSpec((1, 1, _C), lambda e: (jnp.minimum(e, _ES - 1), 0, 0)),
            pl.BlockSpec((4, _C, _W), lambda e: (e, 0, 0)),
            pl.BlockSpec((4, 1, _W), lambda e: (e, 0, 0)),
            pl.BlockSpec((4, _W, _C), lambda e: (e, 0, 0)),
            pl.BlockSpec((4, 1, _C), lambda e: (e, 0, 0)),
        ],
        out_specs=pl.BlockSpec((_T, _C), lambda e: (0, 0)),
        out_shape=jax.ShapeDtypeStruct((_T, _C), jnp.float32),
        scratch_shapes=[pltpu.VMEM((_T, _ER), jnp.float32)],
        compiler_params=pltpu.CompilerParams(
            dimension_semantics=("arbitrary",),
        ),
    )(
        u2, centroids,
        shared_g.reshape(1, _C), routed_g.reshape(1, _C),
        shared_W1, shared_b1.reshape(_ES, 1, _W),
        shared_W2, shared_b2.reshape(_ES, 1, _C),
        routed_W1, routed_b1.reshape(_ER, 1, _W),
        routed_W2, routed_b2.reshape(_ER, 1, _C),
    )
    return out.reshape(_B, _T, _C)


# speedup vs baseline: 1.5219x; 1.0119x over previous
import functools
import jax
import jax.numpy as jnp
from jax.experimental import pallas as pl
from jax.experimental.pallas import tpu as pltpu

_B, _T, _C = 1, 512, 256
_W = 512
_ER, _ES, _K = 16, 2, 2
_EPS = 1.1920929e-07


def _rms(x, g):
    return x * jax.lax.rsqrt(jnp.mean(x * x, axis=-1, keepdims=True) + _EPS) * g


def _gelu(x):
    return 0.5 * x * (1.0 + jax.lax.erf(x * 0.7071067811865476))


def _dense_body(u_ref, cent_ref, sg_ref, rg_ref,
                sW1_ref, sb1_ref, sW2_ref, sb2_ref,
                rW1_ref, rb1_ref, rW2_ref, rb2_ref,
                out_ref, g_scr):
    e = pl.program_id(0)
    u = u_ref[...]
    ids = jax.lax.broadcasted_iota(jnp.int32, (_T, _ER), 1)

    @pl.when(e == 0)
    def _init():
        s = jax.nn.sigmoid(
            jnp.dot(u, cent_ref[...], preferred_element_type=jnp.float32))
        denom = jnp.sum(s, axis=1, keepdims=True)
        m1 = jnp.max(s, axis=1, keepdims=True)
        i1 = jnp.min(jnp.where(s == m1, ids, _ER), axis=1, keepdims=True)
        s2 = jnp.where(ids == i1, -jnp.inf, s)
        m2 = jnp.max(s2, axis=1, keepdims=True)
        i2 = jnp.min(jnp.where(s2 == m2, ids, _ER), axis=1, keepdims=True)
        gfull = (jnp.where(ids == i1, m1 / denom, 0.0)
                 + jnp.where(ids == i2, m2 / denom, 0.0))
        g_scr[...] = gfull
        out_ref[...] = u

    bf = jnp.bfloat16

    @pl.when(e < _ES)
    def _shared():
        xn = _rms(u, sg_ref[0, :])
        h = _gelu((jnp.dot(xn.astype(bf), sW1_ref[0].astype(bf),
                           preferred_element_type=jnp.float32)
                   + sb1_ref[0]).astype(bf))
        out_ref[...] += (jnp.dot(h, sW2_ref[0].astype(bf),
                                 preferred_element_type=jnp.float32)
                         + sb2_ref[0])

    xn = _rms(u, rg_ref[0, :])
    xnb = xn.astype(bf)
    acc = jnp.zeros((_T, _C), jnp.float32)
    for sub in range(4):
        ee = e * 4 + sub
        h = _gelu((jnp.dot(xnb, rW1_ref[sub].astype(bf),
                           preferred_element_type=jnp.float32)
                   + rb1_ref[sub]).astype(bf))
        y = jnp.dot(h, rW2_ref[sub].astype(bf),
                    preferred_element_type=jnp.float32) + rb2_ref[sub]
        gcol = jnp.sum(jnp.where(ids == ee, g_scr[...], 0.0), axis=1,
                       keepdims=True)
        acc = acc + gcol * y
    out_ref[...] += acc


def kernel(u, shared_W1, shared_b1, shared_W2, shared_b2, shared_g,
           routed_W1, routed_b1, routed_W2, routed_b2, routed_g, centroids):
    u2 = u.reshape(_T, _C)
    out = pl.pallas_call(
        _dense_body,
        grid=(_ER // 4,),
        in_specs=[
            pl.BlockSpec((_T, _C), lambda e: (0, 0)),
            pl.BlockSpec((_C, _ER), lambda e: (0, 0)),
            pl.BlockSpec((1, _C), lambda e: (0, 0)),
            pl.BlockSpec((1, _C), lambda e: (0, 0)),
            pl.BlockSpec((1, _C, _W), lambda e: (jnp.minimum(e, _ES - 1), 0, 0)),
            pl.BlockSpec((1, 1, _W), lambda e: (jnp.minimum(e, _ES - 1), 0, 0)),
            pl.BlockSpec((1, _W, _C), lambda e: (jnp.minimum(e, _ES - 1), 0, 0)),
            pl.BlockSpec((1, 1, _C), lambda e: (jnp.minimum(e, _ES - 1), 0, 0)),
            pl.BlockSpec((4, _C, _W), lambda e: (e, 0, 0)),
            pl.BlockSpec((4, 1, _W), lambda e: (e, 0, 0)),
            pl.BlockSpec((4, _W, _C), lambda e: (e, 0, 0)),
            pl.BlockSpec((4, 1, _C), lambda e: (e, 0, 0)),
        ],
        out_specs=pl.BlockSpec((_T, _C), lambda e: (0, 0)),
        out_shape=jax.ShapeDtypeStruct((_T, _C), jnp.float32),
        scratch_shapes=[pltpu.VMEM((_T, _ER), jnp.float32)],
        compiler_params=pltpu.CompilerParams(
            dimension_semantics=("arbitrary",),
        ),
    )(
        u2, centroids,
        shared_g.reshape(1, _C), routed_g.reshape(1, _C),
        shared_W1, shared_b1.reshape(_ES, 1, _W),
        shared_W2, shared_b2.reshape(_ES, 1, _C),
        routed_W1, routed_b1.reshape(_ER, 1, _W),
        routed_W2, routed_b2.reshape(_ER, 1, _C),
    )
    return out.reshape(_B, _T, _C)
